# baseline (speedup 1.0000x reference)
import jax
import jax.numpy as jnp
from jax import lax
from jax.experimental import pallas as pl
from jax.experimental.pallas import tpu as pltpu
from jax.experimental.pallas import tpu_sc as plsc

N = 10000
E = 320000
D = 128

NC = 2
NS = 16
NW = NC * NS

CHUNK = 128
FULL_CHUNKS = 78
EDGES_PER_W = FULL_CHUNKS * CHUNK
EXTRA_BASE = NW * EDGES_PER_W
EXTRA_PER_SC = (E - EXTRA_BASE) // NC
NP_ = 10240
ROWS_PER_TILE = NP_ // NS


def _sc_body(h_hbm, src_hbm, dst_hbm, acc_out, deg_out,
             src_a, dst_a, src_b, dst_b, rows_a, rows_b, ones_v, deg_v,
             acc_sh, deg_sh,
             sem_ga, sem_gb, sem_sa, sem_sb, sem_da, sem_db,
             sem_g2a, sem_g2b):
    cid = lax.axis_index("c")
    tid = lax.axis_index("s")
    row0 = tid * ROWS_PER_TILE

    @pl.loop(0, CHUNK)
    def _z(r):
        for k in range(D // 16):
            rows_a[r, pl.ds(k * 16, 16)] = jnp.zeros((16,), jnp.float32)

    @pl.loop(0, CHUNK // 16)
    def _o(r):
        ones_v[pl.ds(r * 16, 16)] = jnp.ones((16,), jnp.float32)

    @pl.loop(0, ROWS_PER_TILE // 16)
    def _d(r):
        deg_v[pl.ds(r * 16, 16)] = jnp.zeros((16,), jnp.float32)

    for i in range(ROWS_PER_TILE // CHUNK):
        pltpu.sync_copy(rows_a, acc_sh.at[pl.ds(row0 + i * CHUNK, CHUNK)])
    pltpu.sync_copy(deg_v, deg_sh.at[pl.ds(row0, ROWS_PER_TILE)])

    plsc.subcore_barrier()

    base = (cid * NS + tid) * EDGES_PER_W
    last = FULL_CHUNKS - 1

    pltpu.sync_copy(src_hbm.at[pl.ds(base, CHUNK)], src_a)
    pltpu.sync_copy(dst_hbm.at[pl.ds(base, CHUNK)], dst_a)
    pltpu.async_copy(h_hbm.at[src_a], rows_a, sem_ga)
    pltpu.async_copy(src_hbm.at[pl.ds(base + CHUNK, CHUNK)], src_b, sem_sb)
    pltpu.async_copy(dst_hbm.at[pl.ds(base + CHUNK, CHUNK)], dst_b, sem_db)

    def _step(c, s_p, d_p, r_p, g_p, is_p, id_p, dg_p,
              s_q, d_q, r_q, g_q, is_q, id_q):
        pltpu.make_async_copy(src_hbm.at[pl.ds(base, CHUNK)], s_q, is_q).wait()
        pltpu.make_async_copy(dst_hbm.at[pl.ds(base, CHUNK)], d_q, id_q).wait()
        pltpu.async_copy(h_hbm.at[s_q], r_q, g_q)
        pltpu.make_async_copy(h_hbm.at[s_p], r_p, g_p).wait()
        pltpu.async_copy(ones_v, deg_sh.at[d_p], dg_p, add=True)
        pltpu.sync_copy(r_p, acc_sh.at[d_p], add=True)
        pltpu.make_async_copy(ones_v, deg_sh.at[d_p], dg_p).wait()
        off = base + jnp.minimum(c + 2, last) * CHUNK
        pltpu.async_copy(src_hbm.at[pl.ds(off, CHUNK)], s_p, is_p)
        pltpu.async_copy(dst_hbm.at[pl.ds(off, CHUNK)], d_p, id_p)

    @pl.loop(0, (FULL_CHUNKS - 2) // 2)
    def _pair(k):
        _step(2 * k, src_a, dst_a, rows_a, sem_ga, sem_sa, sem_da, sem_g2a,
              src_b, dst_b, rows_b, sem_gb, sem_sb, sem_db)
        _step(2 * k + 1, src_b, dst_b, rows_b, sem_gb, sem_sb, sem_db, sem_g2b,
              src_a, dst_a, rows_a, sem_ga, sem_sa, sem_da)

    _step(FULL_CHUNKS - 2, src_a, dst_a, rows_a, sem_ga, sem_sa, sem_da,
          sem_g2a, src_b, dst_b, rows_b, sem_gb, sem_sb, sem_db)

    pltpu.make_async_copy(h_hbm.at[src_b], rows_b, sem_gb).wait()
    pltpu.async_copy(ones_v, deg_sh.at[dst_b], sem_g2b, add=True)
    pltpu.sync_copy(rows_b, acc_sh.at[dst_b], add=True)
    pltpu.make_async_copy(ones_v, deg_sh.at[dst_b], sem_g2b).wait()
    pltpu.make_async_copy(src_hbm.at[pl.ds(base, CHUNK)], src_a, sem_sa).wait()
    pltpu.make_async_copy(dst_hbm.at[pl.ds(base, CHUNK)], dst_a, sem_da).wait()

    @pl.when(tid == 0)
    def _extra():
        for t in range(EXTRA_PER_SC // CHUNK):
            off = EXTRA_BASE + cid * EXTRA_PER_SC + t * CHUNK
            pltpu.sync_copy(src_hbm.at[pl.ds(off, CHUNK)], src_a)
            pltpu.sync_copy(dst_hbm.at[pl.ds(off, CHUNK)], dst_a)
            pltpu.async_copy(h_hbm.at[src_a], rows_a, sem_ga).wait()
            pltpu.sync_copy(rows_a, acc_sh.at[dst_a], add=True)
            pltpu.sync_copy(ones_v, deg_sh.at[dst_a], add=True)

    plsc.subcore_barrier()

    for i in range(ROWS_PER_TILE // CHUNK):
        r0 = row0 + i * CHUNK
        rbuf = rows_a if i % 2 == 0 else rows_b
        pltpu.sync_copy(acc_sh.at[pl.ds(r0, CHUNK)], rbuf)

        @pl.when(cid == 0)
        def _():
            pltpu.sync_copy(rbuf, acc_out.at[0, pl.ds(r0, CHUNK)])

        @pl.when(cid == 1)
        def _():
            pltpu.sync_copy(rbuf, acc_out.at[1, pl.ds(r0, CHUNK)])

    pltpu.sync_copy(deg_sh.at[pl.ds(row0, ROWS_PER_TILE)], deg_v)

    @pl.when(cid == 0)
    def _():
        pltpu.sync_copy(deg_v, deg_out.at[0, pl.ds(row0, ROWS_PER_TILE)])

    @pl.when(cid == 1)
    def _():
        pltpu.sync_copy(deg_v, deg_out.at[1, pl.ds(row0, ROWS_PER_TILE)])


@jax.jit
def _sc_aggregate(h, src, dst):
    mesh = plsc.VectorSubcoreMesh(core_axis_name="c", subcore_axis_name="s",
                                  num_cores=NC, num_subcores=NS)
    return pl.kernel(
        _sc_body,
        out_type=[
            jax.ShapeDtypeStruct((NC, NP_, D), jnp.float32),
            jax.ShapeDtypeStruct((NC, NP_), jnp.float32),
        ],
        mesh=mesh,
        scratch_types=[
            pltpu.VMEM((CHUNK,), jnp.int32),
            pltpu.VMEM((CHUNK,), jnp.int32),
            pltpu.VMEM((CHUNK,), jnp.int32),
            pltpu.VMEM((CHUNK,), jnp.int32),
            pltpu.VMEM((CHUNK, D), jnp.float32),
            pltpu.VMEM((CHUNK, D), jnp.float32),
            pltpu.VMEM((CHUNK,), jnp.float32),
            pltpu.VMEM((ROWS_PER_TILE,), jnp.float32),
            pltpu.VMEM_SHARED((NP_, D), jnp.float32),
            pltpu.VMEM_SHARED((NP_,), jnp.float32),
            pltpu.SemaphoreType.DMA,
            pltpu.SemaphoreType.DMA,
            pltpu.SemaphoreType.DMA,
            pltpu.SemaphoreType.DMA,
            pltpu.SemaphoreType.DMA,
            pltpu.SemaphoreType.DMA,
            pltpu.SemaphoreType.DMA,
            pltpu.SemaphoreType.DMA,
        ],
    )(h, src, dst)


def _tc_pre_body(h_ref, w_ref, b_ref, o_ref):
    o_ref[...] = (jnp.dot(h_ref[...], w_ref[...],
                          preferred_element_type=jnp.float32) + b_ref[...])


@jax.jit
def _tc_pre(h, W1, b):
    R = 2000
    return pl.pallas_call(
        _tc_pre_body,
        grid=(N // R,),
        in_specs=[
            pl.BlockSpec((R, D), lambda i: (i, 0)),
            pl.BlockSpec((D, D), lambda i: (0, 0)),
            pl.BlockSpec((1, D), lambda i: (0, 0)),
        ],
        out_specs=pl.BlockSpec((R, D), lambda i: (i, 0)),
        out_shape=jax.ShapeDtypeStruct((N, D), jnp.float32),
    )(h, W1, b)


def _tc_body(p_ref, s_ref, d_ref, w_ref, g_ref, be_ref, o_ref):
    s = s_ref[0] + s_ref[1]
    ah = s / jnp.maximum(d_ref[...], 1.0)
    x = p_ref[...] + jnp.dot(ah, w_ref[...], preferred_element_type=jnp.float32)
    m = jnp.mean(x, axis=-1, keepdims=True)
    v = jnp.mean((x - m) * (x - m), axis=-1, keepdims=True)
    o_ref[...] = (x - m) * lax.rsqrt(v + 1e-5) * g_ref[...] + be_ref[...]


@jax.jit
def _tc_combine(pre, acc, deg, W2, gamma, beta):
    R = 2000
    grid = (N // R,)
    return pl.pallas_call(
        _tc_body,
        grid=grid,
        in_specs=[
            pl.BlockSpec((R, D), lambda i: (i, 0)),
            pl.BlockSpec((NC, R, D), lambda i: (0, i, 0)),
            pl.BlockSpec((R, 1), lambda i: (i, 0)),
            pl.BlockSpec((D, D), lambda i: (0, 0)),
            pl.BlockSpec((1, D), lambda i: (0, 0)),
            pl.BlockSpec((1, D), lambda i: (0, 0)),
        ],
        out_specs=pl.BlockSpec((R, D), lambda i: (i, 0)),
        out_shape=jax.ShapeDtypeStruct((N, D), jnp.float32),
    )(pre, acc, deg, W2, gamma, beta)


def kernel(h, edge_index, W, b, gamma, beta):
    src = edge_index[0]
    dst = edge_index[1]
    acc, deg = _sc_aggregate(h, src, dst)
    pre = _tc_pre(h, W[0:D, :], b.reshape(1, D))
    degt = (deg[0] + deg[1]).reshape(NP_, 1)
    return _tc_combine(pre, acc, degt, W[D:2 * D, :],
                       gamma.reshape(1, D), beta.reshape(1, D))


# speedup vs baseline: 1.1864x; 1.1864x over previous
import jax
import jax.numpy as jnp
from jax import lax
from jax.experimental import pallas as pl
from jax.experimental.pallas import tpu as pltpu
from jax.experimental.pallas import tpu_sc as plsc

N = 10000
E = 320000
D = 128

NC = 2
NS = 16
NW = NC * NS

CHUNK = 80
FULL_CHUNKS = 125
EDGES_PER_W = FULL_CHUNKS * CHUNK
NP_ = 10240
ROWS_PER_TILE = NP_ // NS
NB = 4


def _sc_body(h_hbm, src_hbm, dst_hbm, acc_out, deg_out,
         src0, src1, src2, src3, dst0, dst1, dst2, dst3,
         st0, st1, rows0, rows1, rows2, rows3, ones_v, deg_v,
         acc_sh, deg_sh,
         g0, g1, g2, g3, sc0, sc1, sc2, sc3, dg0, dg1, dg2, dg3,
         is0, is1, is2, is3, it0, it1):
    cid = lax.axis_index("c")
    tid = lax.axis_index("s")
    row0 = tid * ROWS_PER_TILE
    srcs = [src0, src1, src2, src3]
    dsts = [dst0, dst1, dst2, dst3]
    stages = [st0, st1]
    rows = [rows0, rows1, rows2, rows3]
    gs = [g0, g1, g2, g3]
    scs = [sc0, sc1, sc2, sc3]
    dgs = [dg0, dg1, dg2, dg3]
    iss = [is0, is1, is2, is3]
    its = [it0, it1]

    @pl.loop(0, CHUNK)
    def _z(r):
        for k in range(D // 16):
            rows0[r, pl.ds(k * 16, 16)] = jnp.zeros((16,), jnp.float32)

    @pl.loop(0, CHUNK // 16)
    def _o(r):
        ones_v[pl.ds(r * 16, 16)] = jnp.ones((16,), jnp.float32)

    @pl.loop(0, ROWS_PER_TILE // 16)
    def _d(r):
        deg_v[pl.ds(r * 16, 16)] = jnp.zeros((16,), jnp.float32)

    for i in range(ROWS_PER_TILE // CHUNK):
        pltpu.sync_copy(rows0, acc_sh.at[pl.ds(row0 + i * CHUNK, CHUNK)])
    pltpu.sync_copy(deg_v, deg_sh.at[pl.ds(row0, ROWS_PER_TILE)])

    plsc.subcore_barrier()

    base = (cid * NS + tid) * EDGES_PER_W

    def off(c):
        return base + c * CHUNK

    def pds(c):
        return pl.ds(off(c), CHUNK)

    def step(c, j, wait_sc2, start_gather, do_loads):
        j2, j3 = (j + 2) % NB, (j + 3) % NB
        p2 = (j + 2) % 2
        p3 = (j + 3) % 2
        pltpu.make_async_copy(h_hbm.at[srcs[j]], rows[j], gs[j]).wait()
        if wait_sc2:
            pltpu.make_async_copy(rows[j2], acc_sh.at[dsts[j2]],
                                  scs[j2]).wait()
            pltpu.make_async_copy(ones_v, deg_sh.at[dsts[j2]],
                                  dgs[j2]).wait()
        if start_gather:
            pltpu.make_async_copy(dst_hbm.at[pds(0)], stages[p2],
                                  its[p2]).wait()
            for kk in range(CHUNK // 16):
                dsts[j2][pl.ds(kk * 16, 16)] = stages[p2][pl.ds(kk * 16, 16)]
            pltpu.make_async_copy(src_hbm.at[pds(0)], srcs[j2],
                                  iss[j2]).wait()
            pltpu.async_copy(h_hbm.at[srcs[j2]], rows[j2], gs[j2])
        pltpu.async_copy(rows[j], acc_sh.at[dsts[j]], scs[j], add=True)
        pltpu.async_copy(ones_v, deg_sh.at[dsts[j]], dgs[j], add=True)
        if do_loads:
            pltpu.async_copy(dst_hbm.at[pds(c + 3)], stages[p3], its[p3])
            pltpu.async_copy(src_hbm.at[pds(c + 3)], srcs[j3], iss[j3])

    pltpu.sync_copy(src_hbm.at[pds(0)], src0)
    pltpu.sync_copy(dst_hbm.at[pds(0)], dst0)
    pltpu.async_copy(h_hbm.at[src0], rows0, g0)
    pltpu.sync_copy(src_hbm.at[pds(1)], src1)
    pltpu.sync_copy(dst_hbm.at[pds(1)], dst1)
    pltpu.async_copy(h_hbm.at[src1], rows1, g1)
    pltpu.async_copy(src_hbm.at[pds(2)], src2, is2)
    pltpu.async_copy(dst_hbm.at[pds(2)], st0, it0)

    step(0, 0, False, True, True)
    step(1, 1, False, True, True)

    @pl.loop(0, 30)
    def _quad(k):
        c = 2 + 4 * k
        step(c + 0, 2, True, True, True)
        step(c + 1, 3, True, True, True)
        step(c + 2, 0, True, True, True)
        step(c + 3, 1, True, True, True)

    step(122, 2, True, True, False)
    step(123, 3, True, False, False)
    step(124, 0, True, False, False)
    pltpu.make_async_copy(rows3, acc_sh.at[dst3], sc3).wait()
    pltpu.make_async_copy(ones_v, deg_sh.at[dst3], dg3).wait()
    pltpu.make_async_copy(rows0, acc_sh.at[dst0], sc0).wait()
    pltpu.make_async_copy(ones_v, deg_sh.at[dst0], dg0).wait()

    plsc.subcore_barrier()

    for i in range(ROWS_PER_TILE // CHUNK):
        r0 = row0 + i * CHUNK
        rbuf = rows[i % 2]
        pltpu.sync_copy(acc_sh.at[pl.ds(r0, CHUNK)], rbuf)

        @pl.when(cid == 0)
        def _():
            pltpu.sync_copy(rbuf, acc_out.at[0, pl.ds(r0, CHUNK)])

        @pl.when(cid == 1)
        def _():
            pltpu.sync_copy(rbuf, acc_out.at[1, pl.ds(r0, CHUNK)])

    pltpu.sync_copy(deg_sh.at[pl.ds(row0, ROWS_PER_TILE)], deg_v)

    @pl.when(cid == 0)
    def _():
        pltpu.sync_copy(deg_v, deg_out.at[0, pl.ds(row0, ROWS_PER_TILE)])

    @pl.when(cid == 1)
    def _():
        pltpu.sync_copy(deg_v, deg_out.at[1, pl.ds(row0, ROWS_PER_TILE)])


@jax.jit
def _sc_aggregate(h, src, dst):
    mesh = plsc.VectorSubcoreMesh(core_axis_name="c", subcore_axis_name="s",
                                  num_cores=NC, num_subcores=NS)
    return pl.kernel(
        _sc_body,
        out_type=[
            jax.ShapeDtypeStruct((NC, NP_, D), jnp.float32),
            jax.ShapeDtypeStruct((NC, NP_), jnp.float32),
        ],
        mesh=mesh,
        scratch_types=(
            [pltpu.VMEM((CHUNK,), jnp.int32)] * 10
            + [pltpu.VMEM((CHUNK, D), jnp.float32)] * 4
            + [
                pltpu.VMEM((CHUNK,), jnp.float32),
                pltpu.VMEM((ROWS_PER_TILE,), jnp.float32),
                pltpu.VMEM_SHARED((NP_, D), jnp.float32),
                pltpu.VMEM_SHARED((NP_,), jnp.float32),
            ]
            + [pltpu.SemaphoreType.DMA] * 18
        ),
    )(h, src, dst)


def _tc_pre_body(h_ref, w_ref, b_ref, o_ref):
    o_ref[...] = (jnp.dot(h_ref[...], w_ref[...],
                          preferred_element_type=jnp.float32) + b_ref[...])


@jax.jit
def _tc_pre(h, W1, b):
    R = 2000
    return pl.pallas_call(
        _tc_pre_body,
        grid=(N // R,),
        in_specs=[
            pl.BlockSpec((R, D), lambda i: (i, 0)),
            pl.BlockSpec((D, D), lambda i: (0, 0)),
            pl.BlockSpec((1, D), lambda i: (0, 0)),
        ],
        out_specs=pl.BlockSpec((R, D), lambda i: (i, 0)),
        out_shape=jax.ShapeDtypeStruct((N, D), jnp.float32),
    )(h, W1, b)


def _tc_body(p_ref, s_ref, d_ref, w_ref, g_ref, be_ref, o_ref):
    s = s_ref[0] + s_ref[1]
    ah = s / jnp.maximum(d_ref[...], 1.0)
    x = p_ref[...] + jnp.dot(ah, w_ref[...], preferred_element_type=jnp.float32)
    m = jnp.mean(x, axis=-1, keepdims=True)
    v = jnp.mean((x - m) * (x - m), axis=-1, keepdims=True)
    o_ref[...] = (x - m) * lax.rsqrt(v + 1e-5) * g_ref[...] + be_ref[...]


@jax.jit
def _tc_combine(pre, acc, deg, W2, gamma, beta):
    R = 2000
    grid = (N // R,)
    return pl.pallas_call(
        _tc_body,
        grid=grid,
        in_specs=[
            pl.BlockSpec((R, D), lambda i: (i, 0)),
            pl.BlockSpec((NC, R, D), lambda i: (0, i, 0)),
            pl.BlockSpec((R, 1), lambda i: (i, 0)),
            pl.BlockSpec((D, D), lambda i: (0, 0)),
            pl.BlockSpec((1, D), lambda i: (0, 0)),
            pl.BlockSpec((1, D), lambda i: (0, 0)),
        ],
        out_specs=pl.BlockSpec((R, D), lambda i: (i, 0)),
        out_shape=jax.ShapeDtypeStruct((N, D), jnp.float32),
    )(pre, acc, deg, W2, gamma, beta)


def kernel(h, edge_index, W, b, gamma, beta):
    src = edge_index[0]
    dst = edge_index[1]
    acc, deg = _sc_aggregate(h, src, dst)
    pre = _tc_pre(h, W[0:D, :], b.reshape(1, D))
    degt = (deg[0] + deg[1]).reshape(NP_, 1)
    return _tc_combine(pre, acc, degt, W[D:2 * D, :],
                       gamma.reshape(1, D), beta.reshape(1, D))
